# baseline (speedup 1.0000x reference)
import functools
import jax
import jax.numpy as jnp
from jax import lax
from jax.experimental import pallas as pl
from jax.experimental.pallas import tpu as pltpu
from jax.experimental.pallas import tpu_sc as plsc

_GDN = lax.GatherDimensionNumbers(
    offset_dims=(), collapsed_slice_dims=(0,), start_index_map=(0,))


def _vgather(vec, idx):
    return lax.gather(
        vec, idx[:, None], _GDN, (1,),
        mode=lax.GatherScatterMode.PROMISE_IN_BOUNDS)


def _sqrt16(x):
    m = jnp.maximum(x, 1e-12)
    i = lax.bitcast_convert_type(m, jnp.int32)
    i = jnp.int32(0x5F3759DF) - lax.shift_right_logical(i, 1)
    y = lax.bitcast_convert_type(i, jnp.float32)
    y = y * (1.5 - 0.5 * m * y * y)
    y = y * (1.5 - 0.5 * m * y * y)
    return x * y


def kernel(diffs, zetas, center_idx):
    B, E, C, F = diffs.shape
    S = zetas.shape[0]
    CF = C * F
    rows = B * E
    d2 = diffs.reshape(rows, CF)
    NC, NS = 2, 16
    NW = NC * NS
    rows_w = rows // NW
    R = 256
    n_chunks = rows_w // R
    mesh = plsc.VectorSubcoreMesh(core_axis_name="c", subcore_axis_name="s")

    @functools.partial(
        pl.kernel, mesh=mesh,
        out_type=jax.ShapeDtypeStruct((rows, S), jnp.float32),
        compiler_params=pltpu.CompilerParams(use_tc_tiling_on_sc=False),
        scratch_types=[
            pltpu.VMEM((R, CF), jnp.float32),
            pltpu.VMEM((R, S), jnp.float32),
            pltpu.VMEM((S,), jnp.float32),
            pltpu.VMEM((S,), jnp.int32),
            pltpu.SemaphoreType.DMA,
            pltpu.SemaphoreType.DMA,
        ],
    )
    def k(d_hbm, z_hbm, ci_hbm, out_hbm, dbuf, obuf, zv, civ, s_in, s_out):
        wid = lax.axis_index("s") * NC + lax.axis_index("c")
        base = wid * rows_w

        pltpu.sync_copy(z_hbm, zv)
        pltpu.sync_copy(ci_hbm, civ)

        zneg = [-jnp.abs(zv[pl.ds(16 * v, 16)]) for v in range(3)]
        cvec = [civ[pl.ds(16 * v, 16)] for v in range(3)]
        iota = lax.iota(jnp.int32, 16)
        quart = lax.shift_right_logical(iota, 2)
        lane = 4 * iota + 3 - 16 * quart

        def chunk(gi, _):
            pltpu.async_copy(
                d_hbm.at[pl.ds(base + gi * R, R)], dbuf, s_in).wait()

            def row_step(r, _):
                qs = [dbuf[r, pl.ds(16 * q, 16)] for q in range(4)]
                r2 = _vgather(qs[0], lane)
                for q in range(1, 4):
                    r2 = jnp.where(quart == q, _vgather(qs[q], lane), r2)
                rt = _sqrt16(r2)
                for v in range(3):
                    w = jnp.exp(zneg[v] * _vgather(rt, cvec[v]))
                    obuf[r, pl.ds(16 * v, 16)] = w
                return 0

            lax.fori_loop(0, R, row_step, 0, unroll=4)

            pltpu.async_copy(
                obuf, out_hbm.at[pl.ds(base + gi * R, R)], s_out).wait()
            return 0

        lax.fori_loop(0, n_chunks, chunk, 0)

    out = k(d2, zetas, center_idx.astype(jnp.int32))
    return out.reshape(B, E, S)


# speedup vs baseline: 1.0963x; 1.0963x over previous
import functools
import jax
import jax.numpy as jnp
from jax import lax
from jax.experimental import pallas as pl
from jax.experimental.pallas import tpu as pltpu
from jax.experimental.pallas import tpu_sc as plsc

_GDN = lax.GatherDimensionNumbers(
    offset_dims=(), collapsed_slice_dims=(0,), start_index_map=(0,))


def _vgather(vec, idx):
    return lax.gather(
        vec, idx[:, None], _GDN, (1,),
        mode=lax.GatherScatterMode.PROMISE_IN_BOUNDS)


def _sqrt16(x):
    m = jnp.maximum(x, 1e-12)
    i = lax.bitcast_convert_type(m, jnp.int32)
    i = jnp.int32(0x5F3759DF) - lax.shift_right_logical(i, 1)
    y = lax.bitcast_convert_type(i, jnp.float32)
    y = y * (1.5 - 0.5 * m * y * y)
    y = y * (1.5 - 0.5 * m * y * y)
    return x * y


def kernel(diffs, zetas, center_idx):
    B, E, C, F = diffs.shape
    S = zetas.shape[0]
    CF = C * F
    rows = B * E
    d2 = diffs.reshape(rows, CF)
    NC, NS = 2, 16
    NW = NC * NS
    rows_w = rows // NW
    R = 256
    n_chunks = rows_w // R
    mesh = plsc.VectorSubcoreMesh(core_axis_name="c", subcore_axis_name="s")

    @functools.partial(
        pl.kernel, mesh=mesh,
        out_type=jax.ShapeDtypeStruct((rows, S), jnp.float32),
        compiler_params=pltpu.CompilerParams(use_tc_tiling_on_sc=False),
        scratch_types=[
            pltpu.VMEM((2, R, CF), jnp.float32),
            pltpu.VMEM((2, R, S), jnp.float32),
            pltpu.VMEM((S,), jnp.float32),
            pltpu.VMEM((S,), jnp.int32),
            pltpu.SemaphoreType.DMA,
            pltpu.SemaphoreType.DMA,
            pltpu.SemaphoreType.DMA,
            pltpu.SemaphoreType.DMA,
        ],
    )
    def k(d_hbm, z_hbm, ci_hbm, out_hbm, dbuf, obuf, zv, civ,
          s_in0, s_in1, s_out0, s_out1):
        wid = lax.axis_index("s") * NC + lax.axis_index("c")
        base = wid * rows_w
        s_in = (s_in0, s_in1)
        s_out = (s_out0, s_out1)

        pltpu.sync_copy(z_hbm, zv)
        pltpu.sync_copy(ci_hbm, civ)

        zneg = [-jnp.abs(zv[pl.ds(16 * v, 16)]) for v in range(3)]
        cvec = [civ[pl.ds(16 * v, 16)] for v in range(3)]
        iota = lax.iota(jnp.int32, 16)
        quart = lax.shift_right_logical(iota, 2)
        lane = 4 * iota + 3 - 16 * quart

        def in_copy(gi, b):
            return pltpu.make_async_copy(
                d_hbm.at[pl.ds(base + gi * R, R)], dbuf.at[b], s_in[b])

        def out_copy(gi, b):
            return pltpu.make_async_copy(
                obuf.at[b], out_hbm.at[pl.ds(base + gi * R, R)], s_out[b])

        in_copy(0, 0).start()
        in_copy(1, 1).start()

        def chunk_pair(g, _):
            for b in range(2):
                gi = 2 * g + b
                in_copy(gi, b).wait()

                @pl.when(gi >= 2)
                def _():
                    out_copy(gi - 2, b).wait()

                def row_step(r, _):
                    qs = [dbuf[b, r, pl.ds(16 * q, 16)] for q in range(4)]
                    r2 = _vgather(qs[0], lane)
                    for q in range(1, 4):
                        r2 = jnp.where(quart == q, _vgather(qs[q], lane), r2)
                    rt = _sqrt16(r2)
                    for v in range(3):
                        w = jnp.exp(zneg[v] * _vgather(rt, cvec[v]))
                        obuf[b, r, pl.ds(16 * v, 16)] = w
                    return 0

                lax.fori_loop(0, R, row_step, 0, unroll=8)

                out_copy(gi, b).start()

                @pl.when(gi + 2 < n_chunks)
                def _():
                    in_copy(gi + 2, b).start()
            return 0

        lax.fori_loop(0, n_chunks // 2, chunk_pair, 0)

        out_copy(n_chunks - 2, 0).wait()
        out_copy(n_chunks - 1, 1).wait()

    out = k(d2, zetas, center_idx.astype(jnp.int32))
    return out.reshape(B, E, S)
